# baseline (speedup 1.0000x reference)
import functools

import jax
import jax.numpy as jnp
from jax import lax
from jax.experimental import pallas as pl
from jax.experimental.pallas import tpu as pltpu
from jax.experimental.pallas import tpu_sc as plsc

N = 10000
E = 320000
R = 8
D = 128
NC = 2
NS = 16
L = 16
C = 80
G = 400
GC = G // C
NB = 3
NO = 2
EPT = E // NS
NG = EPT // G
RPC = R // NC
N2 = 10240
STRIPE = N2 // NS


def _sc_aggregate(src, dst, w, xp):
    mesh = plsc.VectorSubcoreMesh(
        core_axis_name="c", subcore_axis_name="s",
        num_cores=NC, num_subcores=NS)

    @functools.partial(
        pl.kernel,
        out_type=(jax.ShapeDtypeStruct((R, N2, D), jnp.float32),
                  jax.ShapeDtypeStruct((R * NS * N2,), jnp.float32)),
        mesh=mesh,
        compiler_params=pltpu.CompilerParams(needs_layout_passes=False,
                                             use_tc_tiling_on_sc=False),
        scratch_types=[
            pltpu.VMEM((G,), jnp.int32),
            pltpu.VMEM((G,), jnp.int32),
            pltpu.VMEM((G,), jnp.float32),
            pltpu.VMEM((G,), jnp.int32),
            pltpu.VMEM((G,), jnp.int32),
            pltpu.VMEM((G,), jnp.float32),
            pltpu.VMEM((NB, C, D // 2), jnp.int32),
            pltpu.VMEM((NO, C, D), jnp.float32),
            pltpu.VMEM((NO, C), jnp.int32),
            pltpu.VMEM((N2,), jnp.float32),
            pltpu.VMEM_SHARED((N2, D), jnp.float32),
            [pltpu.SemaphoreType.DMA] * NB,
            [pltpu.SemaphoreType.DMA] * NO,
            pltpu.SemaphoreType.DMA,
            pltpu.SemaphoreType.DMA,
        ],
    )
    def agg(src_hbm, dst_hbm, w_hbm, x_hbm, s_out, hstage,
            src_a, dst_a, w_a, src_b, dst_b, w_b, rows, outs, dsts,
            hist, s_sp, gsems, ssems, sem_a, sem_b):
        cid = lax.axis_index("c")
        sid = lax.axis_index("s")
        zero16 = jnp.zeros((L,), jnp.float32)
        ones16 = jnp.full((L,), 1.0, jnp.float32)

        def zero_out0(i, carry):
            for j in range(D // L):
                outs[0, i, pl.ds(j * L, L)] = zero16
            return carry
        lax.fori_loop(0, C, zero_out0, 0)

        def issue_edges(ebase, g, sbuf, dbuf, wbuf, sem):
            eoff = pl.multiple_of(ebase + g * G, 8)
            pltpu.async_copy(src_hbm.at[pl.ds(eoff, G)], sbuf, sem)
            pltpu.async_copy(dst_hbm.at[pl.ds(eoff, G)], dbuf, sem)
            pltpu.async_copy(w_hbm.at[pl.ds(eoff, G)], wbuf, sem)

        def wait_edges(sbuf, dbuf, wbuf, sem):
            pltpu.make_async_copy(src_hbm.at[pl.ds(0, G)], sbuf, sem).wait()
            pltpu.make_async_copy(dst_hbm.at[pl.ds(0, G)], dbuf, sem).wait()
            pltpu.make_async_copy(w_hbm.at[pl.ds(0, G)], wbuf, sem).wait()

        def gather_issue(sbuf, ch, b):
            off = pl.multiple_of(ch * C, C)
            pltpu.async_copy(x_hbm.at[sbuf.at[pl.ds(off, C)]], rows.at[b],
                             gsems[b])

        def gather_wait(b):
            pltpu.make_async_copy(x_hbm.at[pl.ds(0, C)], rows.at[b],
                                  gsems[b]).wait()

        def scatter_wait(o):
            pltpu.make_async_copy(outs.at[o], s_sp.at[dsts.at[o]],
                                  ssems[o]).wait()

        def process(ch, b, o, dbuf, wbuf):
            cbase = ch * C
            gb = rows.at[b]
            ob = outs.at[o]

            @plsc.parallel_loop(0, C, unroll=4)
            def _(e):
                wv = plsc.load_gather(wbuf, [lax.broadcast(cbase + e, (L,))])
                for j in range(D // (2 * L)):
                    xi = gb[e, pl.ds(j * L, L)]
                    xb = plsc.bitcast(xi, jnp.bfloat16)
                    lo, hi = plsc.unpack(xb,
                                         format=plsc.PackFormat.INTERLEAVED)
                    ob[e, pl.ds(j * 2 * L, L)] = lo * wv
                    ob[e, pl.ds(j * 2 * L + L, L)] = hi * wv
            for k in range(C // L):
                idx16 = dbuf[pl.ds(cbase + k * L, L)]
                dsts[o, pl.ds(k * L, L)] = idx16
                plsc.addupdate_scatter(hist, [idx16], ones16)
            pltpu.async_copy(outs.at[o], s_sp.at[dsts.at[o]], ssems[o],
                             add=True)

        def do_group(g, sbuf, dbuf, wbuf, sem, nsbuf, ndbuf, nwbuf, nsem,
                     ebase):
            wait_edges(sbuf, dbuf, wbuf, sem)

            @pl.when(g < NG - 1)
            def _():
                issue_edges(ebase, g + 1, nsbuf, ndbuf, nwbuf, nsem)
            gather_issue(sbuf, 0, 0)
            gather_issue(sbuf, 1, 1)
            for ch in range(GC):
                gather_wait(ch % NB)
                if ch >= NO:
                    scatter_wait(ch % NO)
                process(ch, ch % NB, ch % NO, dbuf, wbuf)
                if ch + 2 < GC:
                    gather_issue(sbuf, ch + 2, (ch + 2) % NB)
            for o in range(NO):
                scatter_wait((GC - NO + o) % NO)

        def relation(rr, carry):
            r = cid * RPC + rr
            ebase = pl.multiple_of(r * E + sid * EPT, 8)
            issue_edges(ebase, 0, src_a, dst_a, w_a, sem_a)

            for k in range(STRIPE // C):
                so = sid * STRIPE + k * C
                pltpu.sync_copy(outs.at[0], s_sp.at[pl.ds(so, C)])

            def hz(i, carry2):
                hist[pl.ds(i * L, L)] = zero16
                return carry2
            lax.fori_loop(0, N2 // L, hz, 0)
            plsc.subcore_barrier()

            def groups(gp, carry2):
                do_group(2 * gp, src_a, dst_a, w_a, sem_a,
                         src_b, dst_b, w_b, sem_b, ebase)
                do_group(2 * gp + 1, src_b, dst_b, w_b, sem_b,
                         src_a, dst_a, w_a, sem_a, ebase)
                return carry2
            lax.fori_loop(0, NG // 2, groups, 0)
            if NG % 2 == 1:
                do_group(NG - 1, src_a, dst_a, w_a, sem_a,
                         src_b, dst_b, w_b, sem_b, ebase)
            hoff = pl.multiple_of((r * NS + sid) * N2, 8)
            pltpu.sync_copy(hist, hstage.at[pl.ds(hoff, N2)])
            plsc.subcore_barrier()

            for k in range(STRIPE // C):
                so = sid * STRIPE + k * C
                pltpu.sync_copy(s_sp.at[pl.ds(so, C)],
                                s_out.at[r, pl.ds(so, C)])

            lax.fori_loop(0, C, zero_out0, 0)
            plsc.subcore_barrier()
            return carry
        lax.fori_loop(0, RPC, relation, 0)

    return agg(src, dst, w, xp)


def _tc_combine(x, s, hst, wrel, skw, skb):
    B = 1024

    def body(x_ref, s_ref, h_ref, w_ref, kw_ref, kb_ref, o_ref):
        acc = jnp.dot(x_ref[...], kw_ref[...],
                      preferred_element_type=jnp.float32) + kb_ref[...]
        cnt = jnp.sum(h_ref[...], axis=1)
        for r in range(R):
            inv = 1.0 / jnp.maximum(cnt[r], 1.0)
            mean = s_ref[r] * inv[:, None]
            acc = acc + jnp.dot(mean, w_ref[r],
                                preferred_element_type=jnp.float32)
        o_ref[...] = acc

    return pl.pallas_call(
        body,
        grid=(N2 // B,),
        in_specs=[
            pl.BlockSpec((B, D), lambda i: (i, 0)),
            pl.BlockSpec((R, B, D), lambda i: (0, i, 0)),
            pl.BlockSpec((R, NS, B), lambda i: (0, 0, i)),
            pl.BlockSpec((R, D, D), lambda i: (0, 0, 0)),
            pl.BlockSpec((D, D), lambda i: (0, 0)),
            pl.BlockSpec((1, D), lambda i: (0, 0)),
        ],
        out_specs=pl.BlockSpec((B, D), lambda i: (i, 0)),
        out_shape=jax.ShapeDtypeStruct((N2, D), jnp.float32),
    )(x, s, hst, wrel, skw, skb)


def kernel(node_feats, edge_index, edge_weight, rel_fcs, skip_w, skip_b):
    src = edge_index[:, 0, :].reshape(-1)
    dst = edge_index[:, 1, :].reshape(-1)
    xp = lax.bitcast_convert_type(
        (node_feats.astype(jnp.bfloat16)
         .reshape(N, D // 32, 2, 16).transpose(0, 1, 3, 2)
         .reshape(N, D // 2, 2)), jnp.int32)
    s, hst = _sc_aggregate(src, dst, edge_weight.reshape(-1), xp)
    x_pad = jnp.pad(node_feats, ((0, N2 - N), (0, 0)))
    out = _tc_combine(x_pad, s, hst.reshape(R, NS, N2), rel_fcs, skip_w,
                      skip_b.reshape(1, D))
    return out[:N]
